# baseline (speedup 1.0000x reference)
import functools

import jax
import jax.numpy as jnp
from jax import lax
from jax.experimental import pallas as pl
from jax.experimental.pallas import tpu as pltpu
from jax.experimental.pallas import tpu_sc as plsc

_ROWS = 12288
_HW = 1024
_K = 256
_NW = 32
_RPW = _ROWS // _NW
_CH = 16
_NCH = _RPW // _CH
_HPAD = 257
_ABS = 0x7FFFFFFF

_LEVELS = (
    (23, 0xFF, 256, None),
    (15, 0xFF, 256, 23),
    (7, 0xFF, 256, 15),
    (0, 0x7F, 128, 7),
)


def _sc_body(x_hbm, tau_hbm, out_hbm, xs, xt, ov, hist, pfxm, tausc,
             sem_in, sem_out):
    wid = lax.axis_index("c") * 16 + lax.axis_index("s")
    lanes = lax.broadcasted_iota(jnp.int32, (16,), 0)
    hbase = lanes * _HPAD
    ones = jnp.ones((16,), jnp.int32)
    zeros16 = jnp.zeros((16,), jnp.int32)

    pltpu.sync_copy(tau_hbm, tausc)
    tauv = tausc[...]
    tau1m = 1.0 - tauv

    def svec(r):
        return lanes * 16 + ((r + lanes) & 15)

    def rid(k):
        return (lanes - k) & 15

    def chunk_body(g, carry):
        base_row = wid * _RPW + g * _CH

        copies = [
            pltpu.async_copy(
                x_hbm.at[pl.ds((base_row + r) * _HW, _HW)],
                xs.at[pl.ds(r * _HW, _HW)],
                sem_in,
            )
            for r in range(_CH)
        ]
        for cp in copies:
            cp.wait()

        for r in range(_CH):
            sv = svec(r)

            def tr_body(c0, c, sv=sv, r=r):
                v = xs[pl.ds(r * _HW + c0 * 16, 16)]
                plsc.store_scatter(xt, [sv + c0 * 256], v)
                return c

            lax.fori_loop(0, _HW // 16, tr_body, 0)

        kvec = jnp.full((16,), _K, jnp.int32)
        pfx = zeros16

        for shift, dmask, bins, pshift in _LEVELS:
            def zero_body(j, c):
                hist[pl.ds(j * 16, 16)] = zeros16
                return c

            lax.fori_loop(0, (_CH * _HPAD) // 16, zero_body, 0)

            if pshift is not None:
                pfxm[...] = pfx
            hpre = [rid(k) * _HPAD for k in range(16)]
            pfxk = (None if pshift is None else
                    [plsc.load_gather(pfxm, [rid(k)]) for k in range(16)])

            def hist_body(eb, c, shift=shift, dmask=dmask, pshift=pshift,
                          hpre=hpre, pfxk=pfxk):
                ebase = eb * 256
                for k in range(16):
                    v = xt[pl.ds(ebase + k * 16, 16)]
                    au = plsc.bitcast(v, jnp.int32) & _ABS
                    d = lax.shift_right_logical(au, shift) & dmask
                    if pshift is None:
                        plsc.addupdate_scatter(hist, [hpre[k] + d], ones)
                    else:
                        m = lax.shift_right_logical(au, pshift) == pfxk[k]
                        plsc.addupdate_scatter(hist, [hpre[k] + d], ones,
                                               mask=m)
                return c

            lax.fori_loop(0, _HW // 16, hist_body, 0)

            def scan_body(i, carry, bins=bins, kvec=kvec):
                s, dig, kp = carry
                for k in range(4):
                    b = (bins - 1) - (i * 4 + k)
                    cnt = plsc.load_gather(hist, [hbase + b])
                    s_new = s + cnt
                    crossed = jnp.logical_and(s < kvec, s_new >= kvec)
                    dig = jnp.where(crossed, b, dig)
                    kp = jnp.where(crossed, kvec - s, kp)
                    s = s_new
                return (s, dig, kp)

            _, dig, kp = lax.fori_loop(
                0, bins // 4, scan_body, (zeros16, zeros16, kvec))
            kvec = kp
            if pshift is None:
                pfx = dig
            elif shift > 0:
                pfx = (pfx << 8) | dig
            else:
                thresh = (pfx << 7) | dig

        for r in range(_CH):
            sv = svec(r)
            thr = thresh[r]

            def mask_body(c0, c, sv=sv, thr=thr, r=r):
                v = plsc.load_gather(xt, [sv + c0 * 256])
                au = plsc.bitcast(v, jnp.int32) & _ABS
                sp = jnp.where(au >= thr, v, jnp.float32(0.0))
                ov[pl.ds(r * _HW + c0 * 16, 16)] = sp * tauv + v * tau1m
                return c

            lax.fori_loop(0, _HW // 16, mask_body, 0)

        pltpu.async_copy(
            ov, out_hbm.at[pl.ds(base_row * _HW, _CH * _HW)], sem_out
        ).wait()
        return carry

    lax.fori_loop(0, _NCH, chunk_body, 0)


_sc_call = functools.partial(
    pl.kernel,
    out_type=jax.ShapeDtypeStruct((_ROWS * _HW,), jnp.float32),
    mesh=plsc.VectorSubcoreMesh(core_axis_name="c", subcore_axis_name="s"),
    scratch_types=[
        pltpu.VMEM((_CH * _HW,), jnp.float32),
        pltpu.VMEM((_CH * _HW,), jnp.float32),
        pltpu.VMEM((_CH * _HW,), jnp.float32),
        pltpu.VMEM((_CH * _HPAD,), jnp.int32),
        pltpu.VMEM((16,), jnp.int32),
        pltpu.VMEM((16,), jnp.float32),
        pltpu.SemaphoreType.DMA,
        pltpu.SemaphoreType.DMA,
    ],
    compiler_params=pltpu.CompilerParams(needs_layout_passes=False),
)(_sc_body)


@jax.jit
def kernel(x, tau):
    n, c, h, w = x.shape
    x_flat = x.reshape(-1)
    tau_arr = jnp.full((16,), tau, jnp.float32)
    out = _sc_call(x_flat, tau_arr)
    return out.reshape(n, c, h, w)


# speedup vs baseline: 1.2673x; 1.2673x over previous
import functools

import jax
import jax.numpy as jnp
from jax import lax
from jax.experimental import pallas as pl
from jax.experimental.pallas import tpu as pltpu
from jax.experimental.pallas import tpu_sc as plsc

_ROWS = 12288
_HW = 1024
_K = 256
_NW = 32
_RPW = _ROWS // _NW
_CH = 16
_NCH = _RPW // _CH
_HPAD = 257
_ABS = 0x7FFFFFFF

_LEVELS = (
    (23, 0xFF, 256, None),
    (15, 0xFF, 256, 23),
    (7, 0xFF, 256, 15),
    (0, 0x7F, 128, 7),
)


def _sc_body(x_hbm, tau_hbm, out_hbm, xs, ov, hist, tausc, sem_in, sem_out):
    wid = lax.axis_index("c") * 16 + lax.axis_index("s")
    lanes = lax.broadcasted_iota(jnp.int32, (16,), 0)
    hbase = lanes * _HPAD
    ones = jnp.ones((16,), jnp.int32)
    zeros16 = jnp.zeros((16,), jnp.int32)

    pltpu.sync_copy(tau_hbm, tausc)
    tauv = tausc[...]
    tau1m = 1.0 - tauv

    def chunk_body(g, carry):
        base = (wid * _RPW + g * _CH) * _HW

        pltpu.async_copy(
            x_hbm.at[pl.ds(base, _CH * _HW)], xs, sem_in
        ).wait()

        kvec = jnp.full((16,), _K, jnp.int32)
        pfx = zeros16

        for shift, dmask, bins, pshift in _LEVELS:
            def zero_body(j, c):
                for k in range(16):
                    hist[pl.ds(j * 256 + k * 16, 16)] = zeros16
                return c

            lax.fori_loop(0, 16, zero_body, 0)
            hist[pl.ds(4096, 16)] = zeros16

            pfxs = (None if pshift is None else
                    [pfx[r] for r in range(_CH)])

            def hist_body(c0, c, shift=shift, dmask=dmask, pshift=pshift,
                          pfxs=pfxs):
                col = c0 * 16
                for r in range(_CH):
                    v = xs[pl.ds(r * _HW + col, 16)]
                    au = plsc.bitcast(v, jnp.int32) & _ABS
                    d = lax.shift_right_logical(au, shift) & dmask
                    if pshift is None:
                        plsc.addupdate_scatter(hist, [d + r * _HPAD], ones)
                    else:
                        m = lax.shift_right_logical(au, pshift) == pfxs[r]
                        plsc.addupdate_scatter(hist, [d + r * _HPAD], ones,
                                               mask=m)
                return c

            lax.fori_loop(0, _HW // 16, hist_body, 0)

            def scan_body(i, carry, bins=bins, kvec=kvec):
                s, dig, kp = carry
                for k in range(4):
                    b = (bins - 1) - (i * 4 + k)
                    cnt = plsc.load_gather(hist, [hbase + b])
                    s_new = s + cnt
                    crossed = jnp.logical_and(s < kvec, s_new >= kvec)
                    dig = jnp.where(crossed, b, dig)
                    kp = jnp.where(crossed, kvec - s, kp)
                    s = s_new
                return (s, dig, kp)

            _, dig, kp = lax.fori_loop(
                0, bins // 4, scan_body, (zeros16, zeros16, kvec))
            kvec = kp
            if pshift is None:
                pfx = dig
            elif shift > 0:
                pfx = (pfx << 8) | dig
            else:
                thresh = (pfx << 7) | dig

        thrs = [thresh[r] for r in range(_CH)]

        def mask_body(c0, c):
            col = c0 * 16
            for r in range(_CH):
                v = xs[pl.ds(r * _HW + col, 16)]
                au = plsc.bitcast(v, jnp.int32) & _ABS
                sp = jnp.where(au >= thrs[r], v, jnp.float32(0.0))
                ov[pl.ds(r * _HW + col, 16)] = sp * tauv + v * tau1m
            return c

        lax.fori_loop(0, _HW // 16, mask_body, 0)

        pltpu.async_copy(
            ov, out_hbm.at[pl.ds(base, _CH * _HW)], sem_out
        ).wait()
        return carry

    lax.fori_loop(0, _NCH, chunk_body, 0)


_sc_call = functools.partial(
    pl.kernel,
    out_type=jax.ShapeDtypeStruct((_ROWS * _HW,), jnp.float32),
    mesh=plsc.VectorSubcoreMesh(core_axis_name="c", subcore_axis_name="s"),
    scratch_types=[
        pltpu.VMEM((_CH * _HW,), jnp.float32),
        pltpu.VMEM((_CH * _HW,), jnp.float32),
        pltpu.VMEM((_CH * _HPAD,), jnp.int32),
        pltpu.VMEM((16,), jnp.float32),
        pltpu.SemaphoreType.DMA,
        pltpu.SemaphoreType.DMA,
    ],
    compiler_params=pltpu.CompilerParams(needs_layout_passes=False),
)(_sc_body)


@jax.jit
def kernel(x, tau):
    n, c, h, w = x.shape
    x_flat = x.reshape(-1)
    tau_arr = jnp.full((16,), tau, jnp.float32)
    out = _sc_call(x_flat, tau_arr)
    return out.reshape(n, c, h, w)


# speedup vs baseline: 2.4887x; 1.9638x over previous
import functools

import jax
import jax.numpy as jnp
from jax import lax
from jax.experimental import pallas as pl
from jax.experimental.pallas import tpu as pltpu
from jax.experimental.pallas import tpu_sc as plsc

_ROWS = 12288
_HW = 1024
_K = 256
_NW = 32
_RPW = _ROWS // _NW
_CH = 16
_NCH = _RPW // _CH
_HPAD = 257
_ABS = 0x7FFFFFFF

_LEVELS = (
    (23, 0xFF, 256, None),
    (15, 0xFF, 256, 23),
    (7, 0xFF, 256, 15),
    (0, 0x7F, 128, 7),
)


def _sc_body(x_hbm, tau_hbm, out_hbm, xs, ov, hist, tausc, sem_in, sem_out):
    wid = lax.axis_index("c") * 16 + lax.axis_index("s")
    lanes = lax.broadcasted_iota(jnp.int32, (16,), 0)
    hbase = lanes * _HPAD
    ones = jnp.ones((16,), jnp.int32)
    zeros16 = jnp.zeros((16,), jnp.int32)

    pltpu.sync_copy(tau_hbm, tausc)
    tauv = tausc[...]
    tau1m = 1.0 - tauv

    def chunk_body(g, carry):
        base = (wid * _RPW + g * _CH) * _HW

        pltpu.async_copy(
            x_hbm.at[pl.ds(base, _CH * _HW)], xs, sem_in
        ).wait()

        kvec = jnp.full((16,), _K, jnp.int32)
        pfx = zeros16

        for shift, dmask, bins, pshift in _LEVELS:
            @plsc.parallel_loop(0, 16, unroll=2)
            def _(j):
                for k in range(16):
                    hist[pl.ds(j * 256 + k * 16, 16)] = zeros16

            hist[pl.ds(4096, 16)] = zeros16

            pfxs = (None if pshift is None else
                    [pfx[r] for r in range(_CH)])

            @plsc.parallel_loop(0, _HW // 16, unroll=2)
            def _(c0):
                col = c0 * 16
                for r in range(_CH):
                    v = xs[pl.ds(r * _HW + col, 16)]
                    au = plsc.bitcast(v, jnp.int32) & _ABS
                    d = lax.shift_right_logical(au, shift) & dmask
                    if pshift is None:
                        plsc.addupdate_scatter(hist, [d + r * _HPAD], ones)
                    else:
                        m = lax.shift_right_logical(au, pshift) == pfxs[r]
                        plsc.addupdate_scatter(hist, [d + r * _HPAD], ones,
                                               mask=m)

            def scan_body(i, carry):
                s, dig, kp = carry
                for k in range(4):
                    b = (bins - 1) - (i * 4 + k)
                    cnt = plsc.load_gather(hist, [hbase + b])
                    s_new = s + cnt
                    crossed = jnp.logical_and(s < kvec, s_new >= kvec)
                    dig = jnp.where(crossed, b, dig)
                    kp = jnp.where(crossed, kvec - s, kp)
                    s = s_new
                return (s, dig, kp)

            _, dig, kp = plsc.parallel_loop(
                0, bins // 4, unroll=2,
                carry=(zeros16, zeros16, kvec))(scan_body)
            kvec = kp
            if pshift is None:
                pfx = dig
            elif shift > 0:
                pfx = (pfx << 8) | dig
            else:
                thresh = (pfx << 7) | dig

        thrs = [thresh[r] for r in range(_CH)]

        @plsc.parallel_loop(0, _HW // 16, unroll=2)
        def _(c0):
            col = c0 * 16
            for r in range(_CH):
                v = xs[pl.ds(r * _HW + col, 16)]
                au = plsc.bitcast(v, jnp.int32) & _ABS
                sp = jnp.where(au >= thrs[r], v, jnp.float32(0.0))
                ov[pl.ds(r * _HW + col, 16)] = sp * tauv + v * tau1m

        pltpu.async_copy(
            ov, out_hbm.at[pl.ds(base, _CH * _HW)], sem_out
        ).wait()
        return carry

    lax.fori_loop(0, _NCH, chunk_body, 0)


_sc_call = functools.partial(
    pl.kernel,
    out_type=jax.ShapeDtypeStruct((_ROWS * _HW,), jnp.float32),
    mesh=plsc.VectorSubcoreMesh(core_axis_name="c", subcore_axis_name="s"),
    scratch_types=[
        pltpu.VMEM((_CH * _HW,), jnp.float32),
        pltpu.VMEM((_CH * _HW,), jnp.float32),
        pltpu.VMEM((_CH * _HPAD,), jnp.int32),
        pltpu.VMEM((16,), jnp.float32),
        pltpu.SemaphoreType.DMA,
        pltpu.SemaphoreType.DMA,
    ],
    compiler_params=pltpu.CompilerParams(needs_layout_passes=False),
)(_sc_body)


@jax.jit
def kernel(x, tau):
    n, c, h, w = x.shape
    x_flat = x.reshape(-1)
    tau_arr = jnp.full((16,), tau, jnp.float32)
    out = _sc_call(x_flat, tau_arr)
    return out.reshape(n, c, h, w)
